# baseline (speedup 1.0000x reference)
import functools

import jax
import jax.numpy as jnp
from jax import lax
from jax.experimental import pallas as pl
from jax.experimental.pallas import tpu as pltpu
from jax.experimental.pallas import tpu_sc as plsc

N = 10000
E = 320000
B = 400
A = 5
D = 128

NC = 2
NS = 16
L = 16

NR = 20
NW = 512
SH = 9
MSK = NW - 1
NCH = NR * NW // L
NP = NR * NW

EPT = E // NS
EV = EPT // L
BCH = B // L

_MESH = plsc.VectorSubcoreMesh(
    core_axis_name="c", subcore_axis_name="s", num_cores=NC, num_subcores=NS)

_PARAMS = pltpu.CompilerParams(use_tc_tiling_on_sc=False,
                               needs_layout_passes=False)


def _zero2d(ref):
    def body(k, _):
        ref[k >> 5, pl.ds((k & 31) * L, L)] = jnp.zeros((L,), jnp.float32)
        return 0
    lax.fori_loop(0, NCH, body, 0)


def _rsqrt(d):
    i = plsc.bitcast(d, jnp.int32)
    y = plsc.bitcast(jnp.int32(0x5F3759DF) - (i >> 1), jnp.float32)
    for _ in range(3):
        y = y * (1.5 - 0.5 * d * y * y)
    return y


def _rbf(x):
    u = plsc.bitcast(x, jnp.int32)
    r = (u + jnp.int32(0x7FFF) + ((u >> 16) & 1)) & jnp.int32(-65536)
    return plsc.bitcast(r, jnp.float32)


def _cumsum_y(v_y):
    def cs(i, carry):
        y_v = v_y[pl.ds(i * L, L)]
        inc = plsc.cumsum(y_v)
        v_y[pl.ds(i * L, L)] = inc - y_v + carry
        return carry + jnp.sum(y_v)
    lax.fori_loop(0, BCH, cs, 0)


def _gather_out(c, s, v_y, v_a, v_b, v_stage, coef):
    def do_chunk(cid):
        idx_v = v_y[pl.ds(cid * L, L)]
        for k in range(6):
            node = idx_v + k
            q, r = node >> SH, node & MSK
            v_stage[k, pl.ds(0, L)] = plsc.load_gather(v_a, [q, r])
            v_stage[6 + k, pl.ds(0, L)] = plsc.load_gather(v_b, [q, r])
        for j in range(12):
            pltpu.sync_copy(v_stage.at[j],
                            coef.at[pl.ds((c * 12 + j) * B + cid * L, L)])

    do_chunk(s)

    @pl.when(s + NS < BCH)
    def _():
        do_chunk(s + NS)


_G_SCRATCH = [
    pltpu.VMEM((EPT,), jnp.int32),
    pltpu.VMEM((EPT,), jnp.int32),
    pltpu.VMEM((EPT,), jnp.float32),
    pltpu.VMEM((NR, NW), jnp.float32),
    pltpu.VMEM((NR, NW), jnp.float32),
    pltpu.VMEM((NR, NW), jnp.float32),
    pltpu.VMEM((NR, NW), jnp.float32),
    pltpu.VMEM((NR, NW), jnp.float32),
    pltpu.VMEM((B,), jnp.int32),
    pltpu.VMEM((NR,), jnp.int32),
    pltpu.VMEM((12, L), jnp.float32),
    pltpu.VMEM((NW,), jnp.float32),
    pltpu.VMEM_SHARED((NR, NW), jnp.float32),
    pltpu.VMEM_SHARED((NR, NW), jnp.float32),
]


@functools.partial(
    pl.kernel,
    out_type=(jax.ShapeDtypeStruct((NC * 12 * B,), jnp.float32),
              jax.ShapeDtypeStruct((NC * 2 * NP,), jnp.float32)),
    mesh=_MESH,
    compiler_params=_PARAMS,
    scratch_types=_G_SCRATCH,
)
def _graph_kernel(xs, rows, cols, ews, ys, iota20, coef, sdv,
                  ve_row, ve_col, ve_ew, v_t, v_dinv, v_tn, v_a, v_b,
                  v_y, v_i20, v_stage, v_zero, sA, sB):
    c = lax.axis_index("c")
    s = lax.axis_index("s")
    ebase = c * E + s * EPT

    pltpu.sync_copy(rows.at[pl.ds(ebase, EPT)], ve_row)
    pltpu.sync_copy(cols.at[pl.ds(ebase, EPT)], ve_col)
    pltpu.sync_copy(ews.at[pl.ds(ebase, EPT)], ve_ew)
    for r in range(NR):
        pltpu.sync_copy(xs.at[pl.ds(c * NP + r * NW, NW)], v_t.at[r])
    pltpu.sync_copy(ys.at[pl.ds(c * B, B)], v_y)
    pltpu.sync_copy(iota20, v_i20)

    def zk(k, _):
        v_zero[pl.ds(k * L, L)] = jnp.zeros((L,), jnp.float32)
        return 0
    lax.fori_loop(0, NW // L, zk, 0)

    def zero_shared(sref):
        pltpu.sync_copy(v_zero, sref.at[s])

        @pl.when(s + NS < NR)
        def _():
            pltpu.sync_copy(v_zero, sref.at[s + NS])

    def col_qr(i):
        col_v = ve_col[pl.ds(i * L, L)]
        return col_v >> SH, col_v & MSK

    def row_qr(i):
        row_v = ve_row[pl.ds(i * L, L)]
        return row_v >> SH, row_v & MSK

    _zero2d(v_a)
    zero_shared(sA)
    plsc.subcore_barrier()

    def ea(i, _):
        qc, rc = col_qr(i)
        plsc.addupdate_scatter(v_a, [qc, rc], ve_ew[pl.ds(i * L, L)])
        return 0
    lax.fori_loop(0, EV, ea, 0)

    pltpu.sync_copy(v_a, sA.at[v_i20], add=True)
    plsc.subcore_barrier()
    pltpu.sync_copy(sA, v_a)

    def pa(k, _):
        r, j = k >> 5, (k & 31) * L
        dv = _rsqrt(v_a[r, pl.ds(j, L)] + 1.0)
        v_dinv[r, pl.ds(j, L)] = dv
        v_t[r, pl.ds(j, L)] = v_t[r, pl.ds(j, L)] * dv
        return 0
    lax.fori_loop(0, NCH, pa, 0)
    plsc.subcore_barrier()

    _zero2d(v_a)
    zero_shared(sA)
    plsc.subcore_barrier()

    def eb(i, _):
        qr, rr = row_qr(i)
        qc, rc = col_qr(i)
        tv = plsc.load_gather(v_t, [qr, rr])
        plsc.addupdate_scatter(v_a, [qc, rc], tv * ve_ew[pl.ds(i * L, L)])
        return 0
    lax.fori_loop(0, EV, eb, 0)

    pltpu.sync_copy(v_a, sA.at[v_i20], add=True)
    plsc.subcore_barrier()
    pltpu.sync_copy(sA, v_a)

    def pb(k, _):
        r, j = k >> 5, (k & 31) * L
        dv = v_dinv[r, pl.ds(j, L)]
        sv = dv * (v_a[r, pl.ds(j, L)] + v_t[r, pl.ds(j, L)])
        v_b[r, pl.ds(j, L)] = sv
        v_t[r, pl.ds(j, L)] = jnp.maximum(sv, 0.0) * dv
        v_tn[r, pl.ds(j, L)] = jnp.maximum(-sv, 0.0) * dv
        return 0
    lax.fori_loop(0, NCH, pb, 0)

    def put_rows(src, base):
        pltpu.sync_copy(src.at[s], sdv.at[pl.ds(base + s * NW, NW)])

        @pl.when(s + NS < NR)
        def _():
            pltpu.sync_copy(src.at[s + NS],
                            sdv.at[pl.ds(base + (s + NS) * NW, NW)])
    put_rows(v_b, c * (2 * NP))
    put_rows(v_dinv, c * (2 * NP) + NP)
    plsc.subcore_barrier()

    _zero2d(v_a)
    _zero2d(v_b)
    zero_shared(sA)
    zero_shared(sB)
    plsc.subcore_barrier()

    def ec(i, _):
        qr, rr = row_qr(i)
        qc, rc = col_qr(i)
        ev = ve_ew[pl.ds(i * L, L)]
        tpv = plsc.load_gather(v_t, [qr, rr])
        tnv = plsc.load_gather(v_tn, [qr, rr])
        plsc.addupdate_scatter(v_a, [qc, rc], tpv * ev)
        plsc.addupdate_scatter(v_b, [qc, rc], tnv * ev)
        return 0
    lax.fori_loop(0, EV, ec, 0)

    pltpu.sync_copy(v_a, sA.at[v_i20], add=True)
    pltpu.sync_copy(v_b, sB.at[v_i20], add=True)
    plsc.subcore_barrier()
    pltpu.sync_copy(sA, v_a)
    pltpu.sync_copy(sB, v_b)

    def pc(k, _):
        r, j = k >> 5, (k & 31) * L
        dv = v_dinv[r, pl.ds(j, L)]
        v_a[r, pl.ds(j, L)] = dv * (v_a[r, pl.ds(j, L)] + v_t[r, pl.ds(j, L)])
        v_b[r, pl.ds(j, L)] = dv * (v_b[r, pl.ds(j, L)] + v_tn[r, pl.ds(j, L)])
        return 0
    lax.fori_loop(0, NCH, pc, 0)

    _cumsum_y(v_y)
    _gather_out(c, s, v_y, v_a, v_b, v_stage, coef)


@functools.partial(
    pl.kernel,
    out_type=jax.ShapeDtypeStruct((NC * 12 * B,), jnp.float32),
    mesh=_MESH,
    compiler_params=_PARAMS,
    scratch_types=_G_SCRATCH,
)
def _proj_kernel(fts, dvs, rows, cols, ews, ys, iota20, gcoef,
                 ve_row, ve_col, ve_ew, v_t, v_dinv, v_tn, v_a, v_b,
                 v_y, v_i20, v_stage, v_zero, sA, sB):
    c = lax.axis_index("c")
    s = lax.axis_index("s")
    ebase = c * E + s * EPT

    pltpu.sync_copy(rows.at[pl.ds(ebase, EPT)], ve_row)
    pltpu.sync_copy(cols.at[pl.ds(ebase, EPT)], ve_col)
    pltpu.sync_copy(ews.at[pl.ds(ebase, EPT)], ve_ew)
    for r in range(NR):
        pltpu.sync_copy(fts.at[pl.ds(c * (2 * NP) + r * NW, NW)], v_t.at[r])
        pltpu.sync_copy(fts.at[pl.ds(c * (2 * NP) + NP + r * NW, NW)],
                        v_tn.at[r])
        pltpu.sync_copy(dvs.at[pl.ds(c * NP + r * NW, NW)], v_dinv.at[r])
    pltpu.sync_copy(ys.at[pl.ds(c * B, B)], v_y)
    pltpu.sync_copy(iota20, v_i20)

    def zk(k, _):
        v_zero[pl.ds(k * L, L)] = jnp.zeros((L,), jnp.float32)
        return 0
    lax.fori_loop(0, NW // L, zk, 0)

    _zero2d(v_a)
    _zero2d(v_b)
    pltpu.sync_copy(v_zero, sA.at[s])
    pltpu.sync_copy(v_zero, sB.at[s])

    @pl.when(s + NS < NR)
    def _():
        pltpu.sync_copy(v_zero, sA.at[s + NS])
        pltpu.sync_copy(v_zero, sB.at[s + NS])
    plsc.subcore_barrier()

    def ec(i, _):
        row_v = ve_row[pl.ds(i * L, L)]
        col_v = ve_col[pl.ds(i * L, L)]
        qr, rr = row_v >> SH, row_v & MSK
        qc, rc = col_v >> SH, col_v & MSK
        ev = ve_ew[pl.ds(i * L, L)]
        f1 = plsc.load_gather(v_t, [qr, rr])
        f2 = plsc.load_gather(v_tn, [qr, rr])
        plsc.addupdate_scatter(v_a, [qc, rc], f1 * ev)
        plsc.addupdate_scatter(v_b, [qc, rc], f2 * ev)
        return 0
    lax.fori_loop(0, EV, ec, 0)

    pltpu.sync_copy(v_a, sA.at[v_i20], add=True)
    pltpu.sync_copy(v_b, sB.at[v_i20], add=True)
    plsc.subcore_barrier()
    pltpu.sync_copy(sA, v_a)
    pltpu.sync_copy(sB, v_b)

    def pc(k, _):
        r, j = k >> 5, (k & 31) * L
        dv = v_dinv[r, pl.ds(j, L)]
        v_a[r, pl.ds(j, L)] = dv * (v_a[r, pl.ds(j, L)] + v_t[r, pl.ds(j, L)])
        v_b[r, pl.ds(j, L)] = dv * (v_b[r, pl.ds(j, L)] + v_tn[r, pl.ds(j, L)])
        return 0
    lax.fori_loop(0, NCH, pc, 0)

    _cumsum_y(v_y)
    _gather_out(c, s, v_y, v_a, v_b, v_stage, gcoef)


_SC = 0.08838834764831845


@functools.partial(
    pl.kernel,
    out_type=(
        jax.ShapeDtypeStruct((B,), jnp.float32),
        jax.ShapeDtypeStruct((A * B,), jnp.float32),
        jax.ShapeDtypeStruct((A * B,), jnp.float32),
        jax.ShapeDtypeStruct((A * B,), jnp.float32),
    ),
    mesh=_MESH,
    compiler_params=_PARAMS,
    scratch_types=[
        pltpu.VMEM((12, L), jnp.float32),
        pltpu.VMEM((12, L), jnp.float32),
        pltpu.VMEM((12, L), jnp.float32),
        pltpu.VMEM((12, L), jnp.float32),
        pltpu.VMEM((4, L), jnp.float32),
        pltpu.VMEM((4, D), jnp.float32),
        pltpu.VMEM((6, L), jnp.float32),
        pltpu.VMEM((6, L), jnp.float32),
        pltpu.VMEM((6, L), jnp.float32),
        pltpu.VMEM((6, L), jnp.float32),
        pltpu.VMEM((L,), jnp.float32),
        pltpu.VMEM((A, L), jnp.float32),
        pltpu.VMEM((A, L), jnp.float32),
        pltpu.VMEM((A, L), jnp.float32),
    ],
)
def _head_kernel(coef, gcoef, gram_b, pq, ov, asp, ua, ia,
                 v_uc, v_ic, v_ug, v_ig, v_g, v_pq,
                 v_ru1, v_ru2, v_ri1, v_ri2, v_ov, v_asp, v_ua, v_ia):
    c = lax.axis_index("c")
    s = lax.axis_index("s")
    wid = s * NC + c

    @pl.when(wid < BCH)
    def _():
        base = wid * L
        for j in range(12):
            pltpu.sync_copy(coef.at[pl.ds(j * B + base, L)], v_uc.at[j])
            pltpu.sync_copy(coef.at[pl.ds((12 + j) * B + base, L)], v_ic.at[j])
            pltpu.sync_copy(gcoef.at[pl.ds(j * B + base, L)], v_ug.at[j])
            pltpu.sync_copy(gcoef.at[pl.ds((12 + j) * B + base, L)],
                            v_ig.at[j])
        pltpu.sync_copy(gram_b, v_g)
        pltpu.sync_copy(pq, v_pq)

        lane_iota = lax.iota(jnp.int32, L)
        zv = jnp.zeros((L,), jnp.float32)

        def lane_body(lane, carry):
            onehot = jnp.where(lane_iota == lane, 1.0, 0.0)
            out = []
            idx = 0
            for side in range(2):
                vc = v_uc if side == 0 else v_ic
                pb_, qb_ = (0, 1) if side == 0 else (2, 3)
                po_, qo_ = (2, 3) if side == 0 else (0, 1)
                for pos in range(6):
                    a_s = jnp.sum(vc[pos, pl.ds(0, L)] * onehot)
                    c_s = jnp.sum(vc[6 + pos, pl.ds(0, L)] * onehot)
                    acc1 = zv
                    acc2 = zv
                    for h in range(D // L):
                        dh = pl.ds(h * L, L)
                        rw = a_s * v_pq[pb_, dh] + c_s * v_pq[qb_, dh]
                        rho = _rbf(rw) - rw
                        acc1 = acc1 + rho * v_pq[po_, dh]
                        acc2 = acc2 + rho * v_pq[qo_, dh]
                    out.append(carry[idx] + onehot * jnp.sum(acc1))
                    out.append(carry[idx + 1] + onehot * jnp.sum(acc2))
                    idx += 2
            return tuple(out)

        rvals = lax.fori_loop(0, L, lane_body, tuple([zv] * 24))
        for pos in range(6):
            v_ru1[pos, pl.ds(0, L)] = rvals[2 * pos]
            v_ru2[pos, pl.ds(0, L)] = rvals[2 * pos + 1]
            v_ri1[pos, pl.ds(0, L)] = rvals[12 + 2 * pos]
            v_ri2[pos, pl.ds(0, L)] = rvals[12 + 2 * pos + 1]

        d0 = pl.ds(0, L)
        gpp, gpq, gqp, gqq = v_g[0, d0], v_g[1, d0], v_g[2, d0], v_g[3, d0]

        AU = [v_uc[x, d0] for x in range(6)]
        CU = [v_uc[6 + x, d0] for x in range(6)]
        AI = [v_ic[x, d0] for x in range(6)]
        CI = [v_ic[6 + x, d0] for x in range(6)]
        GU1 = [v_ug[x, d0] for x in range(6)]
        GU2 = [v_ug[6 + x, d0] for x in range(6)]
        GI1 = [v_ig[x, d0] for x in range(6)]
        GI2 = [v_ig[6 + x, d0] for x in range(6)]
        RU1 = [v_ru1[x, d0] for x in range(6)]
        RU2 = [v_ru2[x, d0] for x in range(6)]
        RI1 = [v_ri1[x, d0] for x in range(6)]
        RI2 = [v_ri2[x, d0] for x in range(6)]

        def dotc(x, y):
            return (AU[x] * AI[y] * gpp + AU[x] * CI[y] * gpq
                    + CU[x] * AI[y] * gqp + CU[x] * CI[y] * gqq
                    + GU1[x] * AI[y] + GU2[x] * CI[y]
                    + AU[x] * GI1[y] + CU[x] * GI2[y])

        def dotb(x, y):
            return (dotc(x, y) + RU1[x] * AI[y] + RU2[x] * CI[y]
                    + AU[x] * RI1[y] + CU[x] * RI2[y])

        siu = [dotb(1 + k, 0) * _SC for k in range(A)]
        sui = [dotb(0, 1 + k) * _SC for k in range(A)]

        def softmax5(scores):
            m = scores[0]
            for k in range(1, A):
                m = jnp.maximum(m, scores[k])
            es = [jnp.exp(sc0 - m) for sc0 in scores]
            tot = es[0]
            for k in range(1, A):
                tot = tot + es[k]
            inv = 1.0 / tot
            return [e * inv for e in es]

        uat = softmax5(siu)
        iat = softmax5(sui)
        uatb = [_rbf(w) for w in uat]
        iatb = [_rbf(w) for w in iat]

        ovv = jnp.zeros((L,), jnp.float32)
        for j in range(A):
            for k in range(A):
                ovv = ovv + iatb[j] * uatb[k] * dotb(1 + j, 1 + k)
        v_ov[d0] = ovv
        for k in range(A):
            v_asp[k, d0] = dotc(1 + k, 1 + k)
            v_ua[k, d0] = uat[k]
            v_ia[k, d0] = iat[k]

        pltpu.sync_copy(v_ov, ov.at[pl.ds(base, L)])
        for k in range(A):
            pltpu.sync_copy(v_asp.at[k], asp.at[pl.ds(k * B + base, L)])
            pltpu.sync_copy(v_ua.at[k], ua.at[pl.ds(k * B + base, L)])
            pltpu.sync_copy(v_ia.at[k], ia.at[pl.ds(k * B + base, L)])


def kernel(U_x, U_edge_index, U_edge_weight, U_y,
           I_x, I_edge_index, I_edge_weight, I_y,
           Wu0, bu0, Wu1, bu1, Wi0, bi0, Wi1, bi1):
    f32 = jnp.float32
    hi = lax.Precision.HIGHEST
    pad = NP - N
    xs = jnp.concatenate([
        jnp.pad(U_x.reshape(N), (0, pad)),
        jnp.pad(I_x.reshape(N), (0, pad)),
    ])
    rows = jnp.concatenate([U_edge_index[0], I_edge_index[0]])
    cols = jnp.concatenate([U_edge_index[1], I_edge_index[1]])
    ews = jnp.concatenate([U_edge_weight, I_edge_weight])
    ys = jnp.concatenate([U_y.reshape(B).astype(jnp.int32),
                          I_y.reshape(B).astype(jnp.int32)])
    iota20 = jnp.arange(NR, dtype=jnp.int32)

    coef, sdv = _graph_kernel(xs, rows, cols, ews, ys, iota20)
    sdv = sdv.reshape(NC, 2, NP)

    W1b_u = Wu1.astype(jnp.bfloat16).astype(f32)
    W1b_i = Wi1.astype(jnp.bfloat16).astype(f32)
    pu = jnp.maximum(Wu0[0], 0.0)
    qu = jnp.maximum(-Wu0[0], 0.0)
    pi_ = jnp.maximum(Wi0[0], 0.0)
    qi = jnp.maximum(-Wi0[0], 0.0)
    Pu = jnp.matmul(pu, W1b_u, precision=hi)
    Qu = jnp.matmul(qu, W1b_u, precision=hi)
    Pi = jnp.matmul(pi_, W1b_i, precision=hi)
    Qi = jnp.matmul(qi, W1b_i, precision=hi)

    def ftab(s_g, dinv_g, p, q, W1b, Po, Qo):
        sp = jnp.maximum(s_g, 0.0)
        sn = jnp.maximum(-s_g, 0.0)
        z = sp[:, None] * p[None, :] + sn[:, None] * q[None, :]
        delta = z.astype(jnp.bfloat16).astype(f32) - z
        v1 = jnp.matmul(W1b, Po, precision=hi)
        v2 = jnp.matmul(W1b, Qo, precision=hi)
        f1 = jnp.matmul(delta, v1, precision=hi) * dinv_g
        f2 = jnp.matmul(delta, v2, precision=hi) * dinv_g
        return f1, f2

    f1u, f2u = ftab(sdv[0, 0], sdv[0, 1], pu, qu, W1b_u, Pi, Qi)
    f1i, f2i = ftab(sdv[1, 0], sdv[1, 1], pi_, qi, W1b_i, Pu, Qu)
    fts = jnp.concatenate([f1u, f2u, f1i, f2i])
    dvs = jnp.concatenate([sdv[0, 1], sdv[1, 1]])

    gcoef = _proj_kernel(fts, dvs, rows, cols, ews, ys, iota20)

    gram = jnp.stack([
        jnp.matmul(Pu, Pi, precision=hi), jnp.matmul(Pu, Qi, precision=hi),
        jnp.matmul(Qu, Pi, precision=hi), jnp.matmul(Qu, Qi, precision=hi),
    ]).astype(f32)
    gram_b = jnp.tile(gram[:, None], (1, L))
    pq = jnp.stack([Pu, Qu, Pi, Qi]).astype(f32)

    ov, asp, ua, ia = _head_kernel(coef, gcoef, gram_b, pq)

    overall = ov
    aspects = asp.reshape(A, B).T
    user_attn = ua.reshape(A, B).T.reshape(B, 1, A)
    item_attn = ia.reshape(A, B).T.reshape(B, 1, A)
    return (overall, aspects, (user_attn, item_attn))
